# Initial kernel scaffold; baseline (speedup 1.0000x reference)
import jax
import jax.numpy as jnp
from jax.experimental import pallas as pl


def kernel(x, router_W, router_b, W1, b1, W2, b2):
    raise NotImplementedError("write your pallas kernel here")


# speedup vs baseline: 2.5389x; 2.5389x over previous
import functools
import math

import jax
import jax.numpy as jnp
from jax import lax
from jax.experimental import pallas as pl
from jax.experimental.pallas import tpu as pltpu
from jax.experimental.pallas import tpu_sc as plsc

B = 1
T = 2048
D = 768
H = 3072
E = 8
CAP = int(math.ceil(T * B / E * 1.25))
N = B * T
SLOTS = E * CAP
ROWS = SLOTS + 8
IDX = 2 * N
TH = 768
TT = 256


def _router_body(x_ref, rwt_ref, rb_ref, slot0_ref, slot1_ref, w0_ref, w1_ref):
    x = x_ref[...]
    logits = jnp.dot(x, rwt_ref[...], preferred_element_type=jnp.float32)
    logits = logits + rb_ref[...]
    m = jnp.max(logits, axis=1, keepdims=True)
    ex = jnp.exp(logits - m)
    g = ex / jnp.sum(ex, axis=1, keepdims=True)

    idx8 = lax.broadcasted_iota(jnp.int32, (N, E), 1)
    m1 = jnp.max(g, axis=1, keepdims=True)
    i1 = jnp.min(jnp.where(g == m1, idx8, E), axis=1, keepdims=True)
    hit1 = idx8 == i1
    g2 = jnp.where(hit1, jnp.float32(-1.0), g)
    m2 = jnp.max(g2, axis=1, keepdims=True)
    i2 = jnp.min(jnp.where(g2 == m2, idx8, E), axis=1, keepdims=True)
    hit2 = idx8 == i2

    mask = (hit1 | hit2).astype(jnp.float32)
    s = mask
    d = 1
    while d < N:
        s = s + jnp.concatenate(
            [jnp.zeros((d, E), jnp.float32), s[:-d, :]], axis=0)
        d *= 2
    pos = s - 1.0
    keep = mask * (pos < CAP).astype(jnp.float32)
    w = g * keep

    def pick(hit, arr):
        return jnp.sum(jnp.where(hit, arr, jnp.zeros_like(arr)),
                       axis=1, keepdims=True)

    w0 = pick(hit1, w)
    w1 = pick(hit2, w)
    k0 = pick(hit1, keep)
    k1 = pick(hit2, keep)
    p0 = pick(hit1, pos).astype(jnp.int32)
    p1 = pick(hit2, pos).astype(jnp.int32)
    slot0_ref[...] = jnp.where(k0 > 0, i1 * CAP + p0, SLOTS)
    slot1_ref[...] = jnp.where(k1 > 0, i2 * CAP + p1, SLOTS)
    w0_ref[...] = w0
    w1_ref[...] = w1


_router = pl.pallas_call(
    _router_body,
    out_shape=[
        jax.ShapeDtypeStruct((N, 1), jnp.int32),
        jax.ShapeDtypeStruct((N, 1), jnp.int32),
        jax.ShapeDtypeStruct((N, 1), jnp.float32),
        jax.ShapeDtypeStruct((N, 1), jnp.float32),
    ],
)


_INV_SQRT2 = 0.7071067811865476


def _ffn_body(xe_ref, w1_ref, b1_ref, w2_ref, b2_ref, out_ref):
    hi = pl.program_id(1)
    xb = xe_ref[...]
    h = jnp.dot(xb, w1_ref[0], preferred_element_type=jnp.float32)
    h = h + b1_ref[0]
    h = 0.5 * h * (1.0 + lax.erf(h * _INV_SQRT2))
    part = jnp.dot(h, w2_ref[0], preferred_element_type=jnp.float32)

    @pl.when(hi == 0)
    def _():
        out_ref[...] = part + b2_ref[0]

    @pl.when(hi != 0)
    def _():
        out_ref[...] = out_ref[...] + part


_ffn = pl.pallas_call(
    _ffn_body,
    grid=(E, H // TH),
    in_specs=[
        pl.BlockSpec((CAP, D), lambda e, h: (e, 0)),
        pl.BlockSpec((1, D, TH), lambda e, h: (e, 0, h)),
        pl.BlockSpec((1, 1, TH), lambda e, h: (e, 0, h)),
        pl.BlockSpec((1, TH, D), lambda e, h: (e, h, 0)),
        pl.BlockSpec((1, 1, D), lambda e, h: (e, 0, 0)),
    ],
    out_specs=pl.BlockSpec((CAP, D), lambda e, h: (e, 0)),
    out_shape=jax.ShapeDtypeStruct((ROWS, D), jnp.float32),
)


_NW = 32
_CHUNK = IDX // _NW


@functools.lru_cache(maxsize=1)
def _sc_kernels():
    mesh = plsc.VectorSubcoreMesh(
        core_axis_name="c", subcore_axis_name="s")
    nc = plsc.get_sparse_core_info().num_cores

    @functools.partial(
        pl.kernel,
        out_type=jax.ShapeDtypeStruct((ROWS, D), jnp.float32),
        mesh=mesh,
        scratch_types=[
            pltpu.VMEM((_CHUNK,), jnp.int32),
            pltpu.VMEM((_CHUNK, D), jnp.float32),
            pltpu.SemaphoreType.DMA,
        ],
    )
    def dispatch(x_hbm, i_hbm, xe_hbm, idx_v, rows_v, sem):
        wid = lax.axis_index("s") * nc + lax.axis_index("c")
        base = wid * _CHUNK
        xbase = base % N
        pltpu.sync_copy(i_hbm.at[pl.ds(base, _CHUNK)], idx_v)
        pltpu.sync_copy(x_hbm.at[pl.ds(xbase, _CHUNK)], rows_v)
        pltpu.async_copy(rows_v, xe_hbm.at[idx_v], sem).wait()

    @functools.partial(
        pl.kernel,
        out_type=jax.ShapeDtypeStruct((IDX, D), jnp.float32),
        mesh=mesh,
        scratch_types=[
            pltpu.VMEM((_CHUNK,), jnp.int32),
            pltpu.VMEM((_CHUNK, D), jnp.float32),
            pltpu.SemaphoreType.DMA,
        ],
    )
    def combine_gather(ob_hbm, i_hbm, yc_hbm, idx_v, rows_v, sem):
        wid = lax.axis_index("s") * nc + lax.axis_index("c")
        base = wid * _CHUNK
        pltpu.sync_copy(i_hbm.at[pl.ds(base, _CHUNK)], idx_v)
        pltpu.async_copy(ob_hbm.at[idx_v], rows_v, sem).wait()
        pltpu.sync_copy(rows_v, yc_hbm.at[pl.ds(base, _CHUNK)])

    return dispatch, combine_gather


def _mix_body(y0_ref, y1_ref, w0_ref, w1_ref, o_ref):
    w0 = w0_ref[...]
    w1 = w1_ref[...]
    y0 = y0_ref[...]
    y1 = y1_ref[...]
    z = jnp.zeros_like(y0)
    o_ref[...] = (jnp.where(w0 > 0, w0 * y0, z)
                  + jnp.where(w1 > 0, w1 * y1, z))


_mix = pl.pallas_call(
    _mix_body,
    grid=(N // TT,),
    in_specs=[
        pl.BlockSpec((TT, D), lambda t: (t, 0)),
        pl.BlockSpec((TT, D), lambda t: (t + N // TT, 0)),
        pl.BlockSpec((TT, 1), lambda t: (t, 0)),
        pl.BlockSpec((TT, 1), lambda t: (t, 0)),
    ],
    out_specs=pl.BlockSpec((TT, D), lambda t: (t, 0)),
    out_shape=jax.ShapeDtypeStruct((N, D), jnp.float32),
)


def kernel(x, router_W, router_b, W1, b1, W2, b2):
    x_flat = x.reshape(N, D)
    slot0, slot1, w0, w1 = _router(
        x_flat, router_W.T, router_b.reshape(1, E))
    slots = jnp.concatenate([slot0, slot1], axis=0).reshape(IDX)
    dispatch, combine_gather = _sc_kernels()
    xe = dispatch(x_flat, slots)
    outb = _ffn(xe, W1, b1.reshape(E, 1, H), W2, b2.reshape(E, 1, D))
    yc = combine_gather(outb, slots)
    y = _mix(yc, yc, w0, w1)
    return y.reshape(B, T, D)


# speedup vs baseline: 4.9139x; 1.9355x over previous
import functools
import math

import jax
import jax.numpy as jnp
from jax import lax
from jax.experimental import pallas as pl
from jax.experimental.pallas import tpu as pltpu
from jax.experimental.pallas import tpu_sc as plsc

B = 1
T = 2048
D = 768
H = 3072
E = 8
CAP = int(math.ceil(T * B / E * 1.25))
N = B * T
SLOTS = E * CAP
TRASH = 256
ROWS = SLOTS + TRASH
IDX = 2 * N
TH = 768
TT = 256


def _router_body(x_ref, rwt_ref, rb_ref, slot0_ref, slot1_ref, w0_ref, w1_ref):
    x = x_ref[...]
    logits = jnp.dot(x, rwt_ref[...], preferred_element_type=jnp.float32)
    logits = logits + rb_ref[...]
    m = jnp.max(logits, axis=1, keepdims=True)
    ex = jnp.exp(logits - m)
    g = ex / jnp.sum(ex, axis=1, keepdims=True)

    idx8 = lax.broadcasted_iota(jnp.int32, (N, E), 1)
    m1 = jnp.max(g, axis=1, keepdims=True)
    i1 = jnp.min(jnp.where(g == m1, idx8, E), axis=1, keepdims=True)
    hit1 = idx8 == i1
    g2 = jnp.where(hit1, jnp.float32(-1.0), g)
    m2 = jnp.max(g2, axis=1, keepdims=True)
    i2 = jnp.min(jnp.where(g2 == m2, idx8, E), axis=1, keepdims=True)
    hit2 = idx8 == i2

    mask = (hit1 | hit2).astype(jnp.float32)
    s = mask
    d = 1
    while d < N:
        s = s + jnp.concatenate(
            [jnp.zeros((d, E), jnp.float32), s[:-d, :]], axis=0)
        d *= 2
    pos = s - 1.0
    keep = mask * (pos < CAP).astype(jnp.float32)
    w = g * keep

    def pick(hit, arr):
        return jnp.sum(jnp.where(hit, arr, jnp.zeros_like(arr)),
                       axis=1, keepdims=True)

    w0 = pick(hit1, w)
    w1 = pick(hit2, w)
    k0 = pick(hit1, keep)
    k1 = pick(hit2, keep)
    p0 = pick(hit1, pos).astype(jnp.int32)
    p1 = pick(hit2, pos).astype(jnp.int32)
    tid = lax.broadcasted_iota(jnp.int32, (N, 1), 0)
    t0 = SLOTS + ((2 * tid) & (TRASH - 1))
    t1 = SLOTS + ((2 * tid + 1) & (TRASH - 1))
    slot0_ref[...] = jnp.where(k0 > 0, i1 * CAP + p0, t0)
    slot1_ref[...] = jnp.where(k1 > 0, i2 * CAP + p1, t1)
    w0_ref[...] = w0
    w1_ref[...] = w1


_router = pl.pallas_call(
    _router_body,
    out_shape=[
        jax.ShapeDtypeStruct((N, 1), jnp.int32),
        jax.ShapeDtypeStruct((N, 1), jnp.int32),
        jax.ShapeDtypeStruct((N, 1), jnp.float32),
        jax.ShapeDtypeStruct((N, 1), jnp.float32),
    ],
)


_INV_SQRT2 = 0.7071067811865476


def _ffn_body(xe_ref, w1_ref, b1_ref, w2_ref, b2_ref, out_ref):
    hi = pl.program_id(1)
    xb = xe_ref[...].astype(jnp.bfloat16)
    h = jnp.dot(xb, w1_ref[0].astype(jnp.bfloat16),
                preferred_element_type=jnp.float32)
    h = h + b1_ref[0]
    h = 0.5 * h * (1.0 + lax.erf(h * _INV_SQRT2))
    part = jnp.dot(h.astype(jnp.bfloat16), w2_ref[0].astype(jnp.bfloat16),
                   preferred_element_type=jnp.float32)

    @pl.when(hi == 0)
    def _():
        out_ref[...] = part + b2_ref[0]

    @pl.when(hi != 0)
    def _():
        out_ref[...] = out_ref[...] + part


_ffn = pl.pallas_call(
    _ffn_body,
    grid=(E, H // TH),
    in_specs=[
        pl.BlockSpec((CAP, D), lambda e, h: (e, 0)),
        pl.BlockSpec((1, D, TH), lambda e, h: (e, 0, h)),
        pl.BlockSpec((1, 1, TH), lambda e, h: (e, 0, h)),
        pl.BlockSpec((1, TH, D), lambda e, h: (e, h, 0)),
        pl.BlockSpec((1, 1, D), lambda e, h: (e, 0, 0)),
    ],
    out_specs=pl.BlockSpec((CAP, D), lambda e, h: (e, 0)),
    out_shape=jax.ShapeDtypeStruct((ROWS, D), jnp.float32),
)


_NW = 32
_CHUNK = IDX // _NW


@functools.lru_cache(maxsize=1)
def _sc_kernels():
    mesh = plsc.VectorSubcoreMesh(
        core_axis_name="c", subcore_axis_name="s")
    nc = plsc.get_sparse_core_info().num_cores

    @functools.partial(
        pl.kernel,
        out_type=jax.ShapeDtypeStruct((ROWS, D), jnp.float32),
        mesh=mesh,
        scratch_types=[
            pltpu.VMEM((_CHUNK,), jnp.int32),
            pltpu.VMEM((_CHUNK, D), jnp.float32),
            pltpu.SemaphoreType.DMA,
        ],
    )
    def dispatch(x_hbm, i_hbm, xe_hbm, idx_v, rows_v, sem):
        wid = lax.axis_index("s") * nc + lax.axis_index("c")
        base = wid * _CHUNK
        xbase = base % N
        pltpu.sync_copy(i_hbm.at[pl.ds(base, _CHUNK)], idx_v)
        pltpu.sync_copy(x_hbm.at[pl.ds(xbase, _CHUNK)], rows_v)
        pltpu.async_copy(rows_v, xe_hbm.at[idx_v], sem).wait()

    @functools.partial(
        pl.kernel,
        out_type=jax.ShapeDtypeStruct((IDX, D), jnp.float32),
        mesh=mesh,
        scratch_types=[
            pltpu.VMEM((_CHUNK,), jnp.int32),
            pltpu.VMEM((_CHUNK, D), jnp.float32),
            pltpu.SemaphoreType.DMA,
        ],
    )
    def combine_gather(ob_hbm, i_hbm, yc_hbm, idx_v, rows_v, sem):
        wid = lax.axis_index("s") * nc + lax.axis_index("c")
        base = wid * _CHUNK
        pltpu.sync_copy(i_hbm.at[pl.ds(base, _CHUNK)], idx_v)
        pltpu.async_copy(ob_hbm.at[idx_v], rows_v, sem).wait()
        pltpu.sync_copy(rows_v, yc_hbm.at[pl.ds(base, _CHUNK)])

    return dispatch, combine_gather


def _mix_body(y0_ref, y1_ref, w0_ref, w1_ref, o_ref):
    w0 = w0_ref[...]
    w1 = w1_ref[...]
    y0 = y0_ref[...]
    y1 = y1_ref[...]
    z = jnp.zeros_like(y0)
    o_ref[...] = (jnp.where(w0 > 0, w0 * y0, z)
                  + jnp.where(w1 > 0, w1 * y1, z))


_mix = pl.pallas_call(
    _mix_body,
    grid=(N // TT,),
    in_specs=[
        pl.BlockSpec((TT, D), lambda t: (t, 0)),
        pl.BlockSpec((TT, D), lambda t: (t + N // TT, 0)),
        pl.BlockSpec((TT, 1), lambda t: (t, 0)),
        pl.BlockSpec((TT, 1), lambda t: (t, 0)),
    ],
    out_specs=pl.BlockSpec((TT, D), lambda t: (t, 0)),
    out_shape=jax.ShapeDtypeStruct((N, D), jnp.float32),
)


def kernel(x, router_W, router_b, W1, b1, W2, b2):
    x_flat = x.reshape(N, D)
    slot0, slot1, w0, w1 = _router(
        x_flat, router_W.T, router_b.reshape(1, E))
    slots = jnp.concatenate([slot0, slot1], axis=0).reshape(IDX)
    dispatch, combine_gather = _sc_kernels()
    xe = dispatch(x_flat, slots)
    outb = _ffn(xe, W1, b1.reshape(E, 1, H), W2, b2.reshape(E, 1, D))
    yc = combine_gather(outb, slots)
    y = _mix(yc, yc, w0, w1)
    return y.reshape(B, T, D)


# speedup vs baseline: 5.4407x; 1.1072x over previous
import functools
import math

import jax
import jax.numpy as jnp
from jax import lax
from jax.experimental import pallas as pl
from jax.experimental.pallas import tpu as pltpu
from jax.experimental.pallas import tpu_sc as plsc

B = 1
T = 2048
D = 768
H = 3072
E = 8
CAP = int(math.ceil(T * B / E * 1.25))
N = B * T
SLOTS = E * CAP
TRASH = 256
ROWS = SLOTS + TRASH
IDX = 2 * N
TH = 768
TT = 256


def _router_body(x_ref, rwt_ref, rb_ref, slots_ref, wts_ref):
    x = x_ref[...]
    logits = jnp.dot(x, rwt_ref[...], preferred_element_type=jnp.float32)
    logits = logits + rb_ref[...]
    m = jnp.max(logits, axis=1, keepdims=True)
    ex = jnp.exp(logits - m)
    g = ex / jnp.sum(ex, axis=1, keepdims=True)

    idx8 = lax.broadcasted_iota(jnp.int32, (N, E), 1)
    m1 = jnp.max(g, axis=1, keepdims=True)
    i1 = jnp.min(jnp.where(g == m1, idx8, E), axis=1, keepdims=True)
    hit1 = idx8 == i1
    g2 = jnp.where(hit1, jnp.float32(-1.0), g)
    m2 = jnp.max(g2, axis=1, keepdims=True)
    i2 = jnp.min(jnp.where(g2 == m2, idx8, E), axis=1, keepdims=True)
    hit2 = idx8 == i2

    mask = (hit1 | hit2).astype(jnp.float32)
    s = mask
    d = 1
    while d < N:
        s = s + jnp.concatenate(
            [jnp.zeros((d, E), jnp.float32), s[:-d, :]], axis=0)
        d *= 2
    pos = s - 1.0
    keep = mask * (pos < CAP).astype(jnp.float32)
    w = g * keep

    def pick(hit, arr):
        return jnp.sum(jnp.where(hit, arr, jnp.zeros_like(arr)),
                       axis=1, keepdims=True)

    w0 = pick(hit1, w)
    w1 = pick(hit2, w)
    k0 = pick(hit1, keep)
    k1 = pick(hit2, keep)
    p0 = pick(hit1, pos).astype(jnp.int32)
    p1 = pick(hit2, pos).astype(jnp.int32)
    tid = lax.broadcasted_iota(jnp.int32, (N, 1), 0)
    t0 = SLOTS + ((2 * tid) & (TRASH - 1))
    t1 = SLOTS + ((2 * tid + 1) & (TRASH - 1))
    slots_ref[pl.ds(0, N), :] = jnp.where(k0 > 0, i1 * CAP + p0, t0)
    slots_ref[pl.ds(N, N), :] = jnp.where(k1 > 0, i2 * CAP + p1, t1)
    wts_ref[pl.ds(0, N), :] = w0
    wts_ref[pl.ds(N, N), :] = w1


_router = pl.pallas_call(
    _router_body,
    out_shape=[
        jax.ShapeDtypeStruct((IDX, 1), jnp.int32),
        jax.ShapeDtypeStruct((IDX, 1), jnp.float32),
    ],
)


_INV_SQRT2 = 0.7071067811865476


def _ffn_body(xe_ref, w1_ref, b1_ref, w2_ref, b2_ref, out_ref):
    xb = xe_ref[...].astype(jnp.bfloat16)
    h = jnp.dot(xb, w1_ref[0].astype(jnp.bfloat16),
                preferred_element_type=jnp.float32)
    h = h + b1_ref[0]
    h = 0.5 * h * (1.0 + lax.erf(h * _INV_SQRT2))
    part = jnp.dot(h.astype(jnp.bfloat16), w2_ref[0].astype(jnp.bfloat16),
                   preferred_element_type=jnp.float32)
    out_ref[...] = part + b2_ref[0]


_ffn = pl.pallas_call(
    _ffn_body,
    grid=(E,),
    in_specs=[
        pl.BlockSpec((CAP, D), lambda e: (e, 0)),
        pl.BlockSpec((1, D, H), lambda e: (e, 0, 0)),
        pl.BlockSpec((1, 1, H), lambda e: (e, 0, 0)),
        pl.BlockSpec((1, H, D), lambda e: (e, 0, 0)),
        pl.BlockSpec((1, 1, D), lambda e: (e, 0, 0)),
    ],
    out_specs=pl.BlockSpec((CAP, D), lambda e: (e, 0)),
    out_shape=jax.ShapeDtypeStruct((ROWS, D), jnp.float32),
)


_NW = 32
_CHUNK = IDX // _NW


@functools.lru_cache(maxsize=1)
def _sc_kernels():
    mesh = plsc.VectorSubcoreMesh(
        core_axis_name="c", subcore_axis_name="s")
    nc = plsc.get_sparse_core_info().num_cores

    @functools.partial(
        pl.kernel,
        out_type=jax.ShapeDtypeStruct((ROWS, D), jnp.float32),
        mesh=mesh,
        scratch_types=[
            pltpu.VMEM((_CHUNK,), jnp.int32),
            pltpu.VMEM((_CHUNK, D), jnp.float32),
            pltpu.SemaphoreType.DMA,
        ],
    )
    def dispatch(x_hbm, i_hbm, xe_hbm, idx_v, rows_v, sem):
        wid = lax.axis_index("s") * nc + lax.axis_index("c")
        base = wid * _CHUNK
        xbase = base % N
        pltpu.sync_copy(i_hbm.at[pl.ds(base, _CHUNK)], idx_v)
        pltpu.sync_copy(x_hbm.at[pl.ds(xbase, _CHUNK)], rows_v)
        pltpu.async_copy(rows_v, xe_hbm.at[idx_v], sem).wait()

    @functools.partial(
        pl.kernel,
        out_type=jax.ShapeDtypeStruct((IDX, D), jnp.float32),
        mesh=mesh,
        scratch_types=[
            pltpu.VMEM((_CHUNK,), jnp.int32),
            pltpu.VMEM((_CHUNK, D), jnp.float32),
            pltpu.SemaphoreType.DMA,
        ],
    )
    def combine_gather(ob_hbm, i_hbm, yc_hbm, idx_v, rows_v, sem):
        wid = lax.axis_index("s") * nc + lax.axis_index("c")
        base = wid * _CHUNK
        pltpu.sync_copy(i_hbm.at[pl.ds(base, _CHUNK)], idx_v)
        pltpu.async_copy(ob_hbm.at[idx_v], rows_v, sem).wait()
        pltpu.sync_copy(rows_v, yc_hbm.at[pl.ds(base, _CHUNK)])

    return dispatch, combine_gather


def _mix_body(y0_ref, y1_ref, w0_ref, w1_ref, o_ref):
    w0 = w0_ref[...]
    w1 = w1_ref[...]
    y0 = y0_ref[...]
    y1 = y1_ref[...]
    z = jnp.zeros_like(y0)
    o_ref[...] = (jnp.where(w0 > 0, w0 * y0, z)
                  + jnp.where(w1 > 0, w1 * y1, z))


_mix = pl.pallas_call(
    _mix_body,
    grid=(N // TT,),
    in_specs=[
        pl.BlockSpec((TT, D), lambda t: (t, 0)),
        pl.BlockSpec((TT, D), lambda t: (t + N // TT, 0)),
        pl.BlockSpec((TT, 1), lambda t: (t, 0)),
        pl.BlockSpec((TT, 1), lambda t: (t + N // TT, 0)),
    ],
    out_specs=pl.BlockSpec((TT, D), lambda t: (t, 0)),
    out_shape=jax.ShapeDtypeStruct((N, D), jnp.float32),
)


def kernel(x, router_W, router_b, W1, b1, W2, b2):
    x_flat = x.reshape(N, D)
    slots2, wts = _router(x_flat, router_W.T, router_b.reshape(1, E))
    slots = slots2.reshape(IDX)
    dispatch, combine_gather = _sc_kernels()
    xe = dispatch(x_flat, slots)
    outb = _ffn(xe, W1, b1.reshape(E, 1, H), W2, b2.reshape(E, 1, D))
    yc = combine_gather(outb, slots)
    y = _mix(yc, yc, wts, wts)
    return y.reshape(B, T, D)


# speedup vs baseline: 5.5365x; 1.0176x over previous
import functools
import math

import jax
import jax.numpy as jnp
from jax import lax
from jax.experimental import pallas as pl
from jax.experimental.pallas import tpu as pltpu
from jax.experimental.pallas import tpu_sc as plsc

B = 1
T = 2048
D = 768
H = 3072
E = 8
CAP = int(math.ceil(T * B / E * 1.25))
N = B * T
SLOTS = E * CAP
TRASH = 256
ROWS = SLOTS + TRASH
IDX = 2 * N
WL = 128
VL = 16


def _router_body(x_ref, rwt_ref, rb_ref, slots_ref, wts_ref):
    x = x_ref[...]
    logits = jnp.dot(x, rwt_ref[...], preferred_element_type=jnp.float32)
    logits = logits + rb_ref[...]
    m = jnp.max(logits, axis=1, keepdims=True)
    ex = jnp.exp(logits - m)
    g = ex / jnp.sum(ex, axis=1, keepdims=True)

    idx8 = lax.broadcasted_iota(jnp.int32, (N, E), 1)
    m1 = jnp.max(g, axis=1, keepdims=True)
    i1 = jnp.min(jnp.where(g == m1, idx8, E), axis=1, keepdims=True)
    hit1 = idx8 == i1
    g2 = jnp.where(hit1, jnp.float32(-1.0), g)
    m2 = jnp.max(g2, axis=1, keepdims=True)
    i2 = jnp.min(jnp.where(g2 == m2, idx8, E), axis=1, keepdims=True)
    hit2 = idx8 == i2

    mask = (hit1 | hit2).astype(jnp.float32)
    s = mask
    d = 1
    while d < N:
        s = s + jnp.concatenate(
            [jnp.zeros((d, E), jnp.float32), s[:-d, :]], axis=0)
        d *= 2
    pos = s - 1.0
    keep = mask * (pos < CAP).astype(jnp.float32)
    w = g * keep

    def pick(hit, arr):
        return jnp.sum(jnp.where(hit, arr, jnp.zeros_like(arr)),
                       axis=1, keepdims=True)

    w0 = pick(hit1, w)
    w1 = pick(hit2, w)
    k0 = pick(hit1, keep)
    k1 = pick(hit2, keep)
    p0 = pick(hit1, pos).astype(jnp.int32)
    p1 = pick(hit2, pos).astype(jnp.int32)
    tid = lax.broadcasted_iota(jnp.int32, (N, 1), 0)
    t0 = SLOTS + ((2 * tid) & (TRASH - 1))
    t1 = SLOTS + ((2 * tid + 1) & (TRASH - 1))
    slots_ref[pl.ds(0, N), :] = jnp.where(k0 > 0, i1 * CAP + p0, t0)
    slots_ref[pl.ds(N, N), :] = jnp.where(k1 > 0, i2 * CAP + p1, t1)
    wts_ref[pl.ds(0, N), :] = jnp.broadcast_to(w0, (N, WL))
    wts_ref[pl.ds(N, N), :] = jnp.broadcast_to(w1, (N, WL))


_router = pl.pallas_call(
    _router_body,
    out_shape=[
        jax.ShapeDtypeStruct((IDX, 1), jnp.int32),
        jax.ShapeDtypeStruct((IDX, WL), jnp.float32),
    ],
)


_INV_SQRT2 = 0.7071067811865476


def _ffn_body(zin_ref, xe_ref, w1_ref, b1_ref, w2_ref, b2_ref, ws_ref,
              out_ref):
    del zin_ref
    xb = xe_ref[...].astype(jnp.bfloat16)
    h = jnp.dot(xb, w1_ref[0].astype(jnp.bfloat16),
                preferred_element_type=jnp.float32)
    h = h + b1_ref[0]
    h = 0.5 * h * (1.0 + lax.erf(h * _INV_SQRT2))
    part = jnp.dot(h.astype(jnp.bfloat16), w2_ref[0].astype(jnp.bfloat16),
                   preferred_element_type=jnp.float32)
    out_ref[...] = (part + b2_ref[0]) * ws_ref[...][:, 0:1]


_ffn = pl.pallas_call(
    _ffn_body,
    grid=(E,),
    in_specs=[
        pl.BlockSpec(memory_space=pl.ANY),
        pl.BlockSpec((CAP, D), lambda e: (e, 0)),
        pl.BlockSpec((1, D, H), lambda e: (e, 0, 0)),
        pl.BlockSpec((1, 1, H), lambda e: (e, 0, 0)),
        pl.BlockSpec((1, H, D), lambda e: (e, 0, 0)),
        pl.BlockSpec((1, 1, D), lambda e: (e, 0, 0)),
        pl.BlockSpec((CAP, WL), lambda e: (e, 0)),
    ],
    out_specs=pl.BlockSpec((CAP, D), lambda e: (e, 0)),
    out_shape=jax.ShapeDtypeStruct((ROWS, D), jnp.float32),
    input_output_aliases={0: 0},
)


_NW = 32
_CHUNK = IDX // _NW
_TCHUNK = N // _NW


@functools.lru_cache(maxsize=1)
def _sc_kernels():
    mesh = plsc.VectorSubcoreMesh(
        core_axis_name="c", subcore_axis_name="s")
    nc = plsc.get_sparse_core_info().num_cores

    @functools.partial(
        pl.kernel,
        out_type=[
            jax.ShapeDtypeStruct((ROWS, D), jnp.float32),
            jax.ShapeDtypeStruct((ROWS, WL), jnp.float32),
        ],
        mesh=mesh,
        scratch_types=[
            pltpu.VMEM((_CHUNK,), jnp.int32),
            pltpu.VMEM((_CHUNK, D), jnp.float32),
            pltpu.VMEM((_CHUNK, WL), jnp.float32),
            pltpu.SemaphoreType.DMA,
            pltpu.SemaphoreType.DMA,
        ],
    )
    def dispatch(x_hbm, i_hbm, w_hbm, xe_hbm, ws_hbm,
                 idx_v, rows_v, wrows_v, sem, semw):
        wid = lax.axis_index("s") * nc + lax.axis_index("c")
        base = wid * _CHUNK
        xbase = base % N
        pltpu.sync_copy(i_hbm.at[pl.ds(base, _CHUNK)], idx_v)
        pltpu.sync_copy(x_hbm.at[pl.ds(xbase, _CHUNK)], rows_v)
        pltpu.sync_copy(w_hbm.at[pl.ds(base, _CHUNK)], wrows_v)
        cw = pltpu.async_copy(wrows_v, ws_hbm.at[idx_v], semw)
        cx = pltpu.async_copy(rows_v, xe_hbm.at[idx_v], sem)
        cx.wait()
        cw.wait()

    @functools.partial(
        pl.kernel,
        out_type=jax.ShapeDtypeStruct((N, D), jnp.float32),
        mesh=mesh,
        scratch_types=[
            pltpu.VMEM((_TCHUNK,), jnp.int32),
            pltpu.VMEM((_TCHUNK,), jnp.int32),
            pltpu.VMEM((_TCHUNK, D), jnp.float32),
            pltpu.VMEM((_TCHUNK, D), jnp.float32),
            pltpu.SemaphoreType.DMA,
            pltpu.SemaphoreType.DMA,
        ],
    )
    def combine(ob_hbm, i_hbm, y_hbm, idx0_v, idx1_v, r0_v, r1_v, s0, s1):
        wid = lax.axis_index("s") * nc + lax.axis_index("c")
        tb = wid * _TCHUNK
        pltpu.sync_copy(i_hbm.at[pl.ds(tb, _TCHUNK)], idx0_v)
        pltpu.sync_copy(i_hbm.at[pl.ds(N + tb, _TCHUNK)], idx1_v)
        c0 = pltpu.async_copy(ob_hbm.at[idx0_v], r0_v, s0)
        c1 = pltpu.async_copy(ob_hbm.at[idx1_v], r1_v, s1)
        c0.wait()
        c1.wait()

        @pl.loop(0, _TCHUNK)
        def _(t):
            for j in range(D // VL):
                sl = pl.ds(j * VL, VL)
                r0_v[t, sl] = r0_v[t, sl] + r1_v[t, sl]

        pltpu.sync_copy(r0_v, y_hbm.at[pl.ds(tb, _TCHUNK)])

    return dispatch, combine


def kernel(x, router_W, router_b, W1, b1, W2, b2):
    x_flat = x.reshape(N, D)
    slots2, wrep = _router(x_flat, router_W.T, router_b.reshape(1, E))
    slots = slots2.reshape(IDX)
    dispatch, combine = _sc_kernels()
    xe, wslot = dispatch(x_flat, slots, wrep)
    zinit = jnp.zeros((ROWS, D), jnp.float32)
    outb = _ffn(zinit, xe, W1, b1.reshape(E, 1, H), W2,
                b2.reshape(E, 1, D), wslot)
    y = combine(outb, slots)
    return y.reshape(B, T, D)


# speedup vs baseline: 5.8167x; 1.0506x over previous
import functools
import math

import jax
import jax.numpy as jnp
from jax import lax
from jax.experimental import pallas as pl
from jax.experimental.pallas import tpu as pltpu
from jax.experimental.pallas import tpu_sc as plsc

B = 1
T = 2048
D = 768
H = 3072
E = 8
CAP = int(math.ceil(T * B / E * 1.25))
N = B * T
SLOTS = E * CAP
TRASH = 256
ROWS = SLOTS + CAP
IDX = 2 * N
WL = 128
VL = 16


def _router_body(x_ref, rw_ref, rb_ref, slots_ref, wts_ref):
    x = x_ref[...]
    logits = lax.dot_general(x, rw_ref[...], (((1,), (1,)), ((), ())),
                             preferred_element_type=jnp.float32)
    logits = logits + rb_ref[...]
    m = jnp.max(logits, axis=1, keepdims=True)
    ex = jnp.exp(logits - m)
    g = ex / jnp.sum(ex, axis=1, keepdims=True)

    idx8 = lax.broadcasted_iota(jnp.int32, (N, E), 1)
    m1 = jnp.max(g, axis=1, keepdims=True)
    i1 = jnp.min(jnp.where(g == m1, idx8, E), axis=1, keepdims=True)
    hit1 = idx8 == i1
    g2 = jnp.where(hit1, jnp.float32(-1.0), g)
    m2 = jnp.max(g2, axis=1, keepdims=True)
    i2 = jnp.min(jnp.where(g2 == m2, idx8, E), axis=1, keepdims=True)
    hit2 = idx8 == i2

    mask = (hit1 | hit2).astype(jnp.float32)
    s = mask
    d = 1
    while d < N:
        s = s + jnp.concatenate(
            [jnp.zeros((d, E), jnp.float32), s[:-d, :]], axis=0)
        d *= 2
    pos = s - 1.0
    keep = mask * (pos < CAP).astype(jnp.float32)
    w = g * keep

    def pick(hit, arr):
        return jnp.sum(jnp.where(hit, arr, jnp.zeros_like(arr)),
                       axis=1, keepdims=True)

    w0 = pick(hit1, w)
    w1 = pick(hit2, w)
    k0 = pick(hit1, keep)
    k1 = pick(hit2, keep)
    p0 = pick(hit1, pos).astype(jnp.int32)
    p1 = pick(hit2, pos).astype(jnp.int32)
    tid = lax.broadcasted_iota(jnp.int32, (N, 1), 0)
    t0 = SLOTS + ((2 * tid) & (TRASH - 1))
    t1 = SLOTS + ((2 * tid + 1) & (TRASH - 1))
    slots_ref[pl.ds(0, N), :] = jnp.where(k0 > 0, i1 * CAP + p0, t0)
    slots_ref[pl.ds(N, N), :] = jnp.where(k1 > 0, i2 * CAP + p1, t1)
    wts_ref[pl.ds(0, N), :] = jnp.broadcast_to(w0, (N, WL))
    wts_ref[pl.ds(N, N), :] = jnp.broadcast_to(w1, (N, WL))


_router = pl.pallas_call(
    _router_body,
    out_shape=[
        jax.ShapeDtypeStruct((IDX, 1), jnp.int32),
        jax.ShapeDtypeStruct((IDX, WL), jnp.float32),
    ],
)


_INV_SQRT2 = 0.7071067811865476


def _ffn_body(xe_ref, w1_ref, b1_ref, w2_ref, b2_ref, ws_ref, out_ref):
    pid = pl.program_id(0)

    @pl.when(pid < E)
    def _():
        xb = xe_ref[...].astype(jnp.bfloat16)
        h = jnp.dot(xb, w1_ref[0].astype(jnp.bfloat16),
                    preferred_element_type=jnp.float32)
        h = h + b1_ref[0]
        h = 0.5 * h * (1.0 + lax.erf(h * _INV_SQRT2))
        part = jnp.dot(h.astype(jnp.bfloat16),
                       w2_ref[0].astype(jnp.bfloat16),
                       preferred_element_type=jnp.float32)
        out_ref[...] = (part + b2_ref[0]) * ws_ref[...][:, 0:1]

    @pl.when(pid == E)
    def _():
        out_ref[...] = jnp.zeros((CAP, D), jnp.float32)


def _clamp_e(e):
    return jnp.minimum(e, E - 1)


_ffn = pl.pallas_call(
    _ffn_body,
    grid=(E + 1,),
    in_specs=[
        pl.BlockSpec((CAP, D), lambda e: (_clamp_e(e), 0)),
        pl.BlockSpec((1, D, H), lambda e: (_clamp_e(e), 0, 0)),
        pl.BlockSpec((1, 1, H), lambda e: (_clamp_e(e), 0, 0)),
        pl.BlockSpec((1, H, D), lambda e: (_clamp_e(e), 0, 0)),
        pl.BlockSpec((1, 1, D), lambda e: (_clamp_e(e), 0, 0)),
        pl.BlockSpec((CAP, WL), lambda e: (_clamp_e(e), 0)),
    ],
    out_specs=pl.BlockSpec((CAP, D), lambda e: (e, 0)),
    out_shape=jax.ShapeDtypeStruct((ROWS, D), jnp.float32),
)


_NW = 32
_CHUNK = IDX // _NW
_TCHUNK = N // _NW


@functools.lru_cache(maxsize=1)
def _sc_kernels():
    mesh = plsc.VectorSubcoreMesh(
        core_axis_name="c", subcore_axis_name="s")
    nc = plsc.get_sparse_core_info().num_cores

    @functools.partial(
        pl.kernel,
        out_type=[
            jax.ShapeDtypeStruct((ROWS, D), jnp.float32),
            jax.ShapeDtypeStruct((ROWS, WL), jnp.float32),
        ],
        mesh=mesh,
        scratch_types=[
            pltpu.VMEM((_CHUNK,), jnp.int32),
            pltpu.VMEM((_CHUNK, D), jnp.float32),
            pltpu.VMEM((_CHUNK, WL), jnp.float32),
            pltpu.SemaphoreType.DMA,
            pltpu.SemaphoreType.DMA,
        ],
    )
    def dispatch(x_hbm, i_hbm, w_hbm, xe_hbm, ws_hbm,
                 idx_v, rows_v, wrows_v, sem, semw):
        wid = lax.axis_index("s") * nc + lax.axis_index("c")
        base = wid * _CHUNK
        xbase = base % N
        pltpu.sync_copy(i_hbm.at[pl.ds(base, _CHUNK)], idx_v)
        pltpu.sync_copy(x_hbm.at[pl.ds(xbase, _CHUNK)], rows_v)
        pltpu.sync_copy(w_hbm.at[pl.ds(base, _CHUNK)], wrows_v)
        cw = pltpu.async_copy(wrows_v, ws_hbm.at[idx_v], semw)
        cx = pltpu.async_copy(rows_v, xe_hbm.at[idx_v], sem)
        cx.wait()
        cw.wait()

    @functools.partial(
        pl.kernel,
        out_type=jax.ShapeDtypeStruct((N, D), jnp.float32),
        mesh=mesh,
        scratch_types=[
            pltpu.VMEM((_TCHUNK,), jnp.int32),
            pltpu.VMEM((_TCHUNK,), jnp.int32),
            pltpu.VMEM((_TCHUNK, D), jnp.float32),
            pltpu.VMEM((_TCHUNK, D), jnp.float32),
            pltpu.SemaphoreType.DMA,
            pltpu.SemaphoreType.DMA,
        ],
    )
    def combine(ob_hbm, i_hbm, y_hbm, idx0_v, idx1_v, r0_v, r1_v, s0, s1):
        wid = lax.axis_index("s") * nc + lax.axis_index("c")
        tb = wid * _TCHUNK
        pltpu.sync_copy(i_hbm.at[pl.ds(tb, _TCHUNK)], idx0_v)
        pltpu.sync_copy(i_hbm.at[pl.ds(N + tb, _TCHUNK)], idx1_v)
        c0 = pltpu.async_copy(ob_hbm.at[idx0_v], r0_v, s0)
        c1 = pltpu.async_copy(ob_hbm.at[idx1_v], r1_v, s1)
        c0.wait()
        c1.wait()

        @pl.loop(0, _TCHUNK)
        def _(t):
            for j in range(D // VL):
                sl = pl.ds(j * VL, VL)
                r0_v[t, sl] = r0_v[t, sl] + r1_v[t, sl]

        pltpu.sync_copy(r0_v, y_hbm.at[pl.ds(tb, _TCHUNK)])

    return dispatch, combine


def kernel(x, router_W, router_b, W1, b1, W2, b2):
    x_flat = x.reshape(N, D)
    slots2, wrep = _router(x_flat, router_W, router_b.reshape(1, E))
    slots = slots2.reshape(IDX)
    dispatch, combine = _sc_kernels()
    xe, wslot = dispatch(x_flat, slots, wrep)
    outb = _ffn(xe, W1, b1.reshape(E, 1, H), W2, b2.reshape(E, 1, D), wslot)
    y = combine(outb, slots)
    return y.reshape(B, T, D)


# speedup vs baseline: 5.9578x; 1.0243x over previous
import functools
import math

import jax
import jax.numpy as jnp
from jax import lax
from jax.experimental import pallas as pl
from jax.experimental.pallas import tpu as pltpu
from jax.experimental.pallas import tpu_sc as plsc

B = 1
T = 2048
D = 768
H = 3072
E = 8
CAP = int(math.ceil(T * B / E * 1.25))
N = B * T
SLOTS = E * CAP
TRASH = 256
ROWS = SLOTS + CAP
IDX = 2 * N
WL = 128
VL = 16


def _router_body(x_ref, rw_ref, rb_ref, slots_ref, wts_ref):
    x = x_ref[...]
    logits = lax.dot_general(x, rw_ref[...], (((1,), (1,)), ((), ())),
                             preferred_element_type=jnp.float32)
    logits = logits + rb_ref[...]
    m = jnp.max(logits, axis=1, keepdims=True)
    ex = jnp.exp(logits - m)
    g = ex / jnp.sum(ex, axis=1, keepdims=True)

    idx8 = lax.broadcasted_iota(jnp.int32, (N, E), 1)
    m1 = jnp.max(g, axis=1, keepdims=True)
    i1 = jnp.min(jnp.where(g == m1, idx8, E), axis=1, keepdims=True)
    hit1 = idx8 == i1
    g2 = jnp.where(hit1, jnp.float32(-1.0), g)
    m2 = jnp.max(g2, axis=1, keepdims=True)
    i2 = jnp.min(jnp.where(g2 == m2, idx8, E), axis=1, keepdims=True)
    hit2 = idx8 == i2

    mask = (hit1 | hit2).astype(jnp.float32)
    s = mask
    d = 1
    while d < N:
        s = s + jnp.concatenate(
            [jnp.zeros((d, E), jnp.float32), s[:-d, :]], axis=0)
        d *= 2
    pos = s - 1.0
    keep = mask * (pos < CAP).astype(jnp.float32)
    w = g * keep

    def pick(hit, arr):
        return jnp.sum(jnp.where(hit, arr, jnp.zeros_like(arr)),
                       axis=1, keepdims=True)

    w0 = pick(hit1, w)
    w1 = pick(hit2, w)
    k0 = pick(hit1, keep)
    k1 = pick(hit2, keep)
    p0 = pick(hit1, pos).astype(jnp.int32)
    p1 = pick(hit2, pos).astype(jnp.int32)
    tid = lax.broadcasted_iota(jnp.int32, (N, 1), 0)
    t0 = SLOTS + ((2 * tid) & (TRASH - 1))
    t1 = SLOTS + ((2 * tid + 1) & (TRASH - 1))
    s0 = jnp.where(k0 > 0, i1 * CAP + p0, t0)
    s1 = jnp.where(k1 > 0, i2 * CAP + p1, t1)
    slots_ref[...] = jnp.concatenate([s0, s1], axis=0).reshape(_NW, _CHUNK)
    wts_ref[pl.ds(0, N), :] = jnp.broadcast_to(w0, (N, WL))
    wts_ref[pl.ds(N, N), :] = jnp.broadcast_to(w1, (N, WL))


_router = pl.pallas_call(
    _router_body,
    out_shape=[
        jax.ShapeDtypeStruct((32, 128), jnp.int32),
        jax.ShapeDtypeStruct((IDX, WL), jnp.float32),
    ],
)


_INV_SQRT2 = 0.7071067811865476


def _ffn_body(xe_ref, w1_ref, b1_ref, w2_ref, b2_ref, ws_ref, out_ref):
    pid = pl.program_id(0)

    @pl.when(pid < E)
    def _():
        xb = xe_ref[...].astype(jnp.bfloat16)
        h = jnp.dot(xb, w1_ref[0].astype(jnp.bfloat16),
                    preferred_element_type=jnp.float32)
        h = h + b1_ref[0]
        h = 0.5 * h * (1.0 + lax.erf(h * _INV_SQRT2))
        part = jnp.dot(h.astype(jnp.bfloat16),
                       w2_ref[0].astype(jnp.bfloat16),
                       preferred_element_type=jnp.float32)
        out_ref[...] = (part + b2_ref[0]) * ws_ref[...][:, 0:1]

    @pl.when(pid == E)
    def _():
        out_ref[...] = jnp.zeros((CAP, D), jnp.float32)


def _clamp_e(e):
    return jnp.minimum(e, E - 1)


_ffn = pl.pallas_call(
    _ffn_body,
    grid=(E + 1,),
    in_specs=[
        pl.BlockSpec((CAP, D), lambda e: (_clamp_e(e), 0)),
        pl.BlockSpec((1, D, H), lambda e: (_clamp_e(e), 0, 0)),
        pl.BlockSpec((1, 1, H), lambda e: (_clamp_e(e), 0, 0)),
        pl.BlockSpec((1, H, D), lambda e: (_clamp_e(e), 0, 0)),
        pl.BlockSpec((1, 1, D), lambda e: (_clamp_e(e), 0, 0)),
        pl.BlockSpec((CAP, WL), lambda e: (_clamp_e(e), 0)),
    ],
    out_specs=pl.BlockSpec((CAP, D), lambda e: (e, 0)),
    out_shape=jax.ShapeDtypeStruct((ROWS, D), jnp.float32),
)


_NW = 32
_CHUNK = IDX // _NW
_TCHUNK = N // _NW


@functools.lru_cache(maxsize=1)
def _sc_kernels():
    mesh = plsc.VectorSubcoreMesh(
        core_axis_name="c", subcore_axis_name="s")
    nc = plsc.get_sparse_core_info().num_cores

    @functools.partial(
        pl.kernel,
        out_type=[
            jax.ShapeDtypeStruct((ROWS, D), jnp.float32),
            jax.ShapeDtypeStruct((ROWS, WL), jnp.float32),
        ],
        mesh=mesh,
        scratch_types=[
            pltpu.VMEM((_CHUNK,), jnp.int32),
            pltpu.VMEM((_CHUNK, D), jnp.float32),
            pltpu.VMEM((_CHUNK, WL), jnp.float32),
            pltpu.SemaphoreType.DMA,
            pltpu.SemaphoreType.DMA,
        ],
    )
    def dispatch(x_hbm, i_hbm, w_hbm, xe_hbm, ws_hbm,
                 idx_v, rows_v, wrows_v, sem, semw):
        wid = lax.axis_index("s") * nc + lax.axis_index("c")
        base = wid * _CHUNK
        xbase = base % N
        pltpu.sync_copy(i_hbm.at[pl.ds(base, _CHUNK)], idx_v)
        pltpu.sync_copy(x_hbm.at[pl.ds(xbase, _CHUNK)], rows_v)
        pltpu.sync_copy(w_hbm.at[pl.ds(base, _CHUNK)], wrows_v)
        cw = pltpu.async_copy(wrows_v, ws_hbm.at[idx_v], semw)
        cx = pltpu.async_copy(rows_v, xe_hbm.at[idx_v], sem)
        cx.wait()
        cw.wait()

    @functools.partial(
        pl.kernel,
        out_type=jax.ShapeDtypeStruct((N, D), jnp.float32),
        mesh=mesh,
        scratch_types=[
            pltpu.VMEM((_TCHUNK,), jnp.int32),
            pltpu.VMEM((_TCHUNK,), jnp.int32),
            pltpu.VMEM((_TCHUNK, D), jnp.float32),
            pltpu.VMEM((_TCHUNK, D), jnp.float32),
            pltpu.SemaphoreType.DMA,
            pltpu.SemaphoreType.DMA,
        ],
    )
    def combine(ob_hbm, i_hbm, y_hbm, idx0_v, idx1_v, r0_v, r1_v, s0, s1):
        wid = lax.axis_index("s") * nc + lax.axis_index("c")
        tb = wid * _TCHUNK
        pltpu.sync_copy(i_hbm.at[pl.ds(tb, _TCHUNK)], idx0_v)
        pltpu.sync_copy(i_hbm.at[pl.ds(N + tb, _TCHUNK)], idx1_v)
        c0 = pltpu.async_copy(ob_hbm.at[idx0_v], r0_v, s0)
        c1 = pltpu.async_copy(ob_hbm.at[idx1_v], r1_v, s1)
        c0.wait()
        c1.wait()

        @pl.loop(0, _TCHUNK)
        def _(t):
            for j in range(D // VL):
                sl = pl.ds(j * VL, VL)
                r0_v[t, sl] = r0_v[t, sl] + r1_v[t, sl]

        pltpu.sync_copy(r0_v, y_hbm.at[pl.ds(tb, _TCHUNK)])

    return dispatch, combine


def kernel(x, router_W, router_b, W1, b1, W2, b2):
    x_flat = x.reshape(N, D)
    slots32, wrep = _router(x_flat, router_W, router_b.reshape(1, E))
    slots = slots32.reshape(IDX)
    dispatch, combine = _sc_kernels()
    xe, wslot = dispatch(x_flat, slots, wrep)
    outb = _ffn(xe, W1, b1.reshape(E, 1, H), W2, b2.reshape(E, 1, D), wslot)
    y = combine(outb, slots)
    return y.reshape(B, T, D)


# speedup vs baseline: 6.1732x; 1.0361x over previous
import functools
import math

import jax
import jax.numpy as jnp
from jax import lax
from jax.experimental import pallas as pl
from jax.experimental.pallas import tpu as pltpu
from jax.experimental.pallas import tpu_sc as plsc

B = 1
T = 2048
D = 768
H = 3072
E = 8
CAP = int(math.ceil(T * B / E * 1.25))
N = B * T
SLOTS = E * CAP
TRASH = 256
ROWS = SLOTS + CAP
IDX = 2 * N
WL = 128
VL = 16


def _router_body(x_ref, rw_ref, rb_ref, slots_ref, wts_ref):
    x = x_ref[...]
    logits = lax.dot_general(x, rw_ref[...], (((1,), (1,)), ((), ())),
                             preferred_element_type=jnp.float32)
    logits = logits + rb_ref[...]
    m = jnp.max(logits, axis=1, keepdims=True)
    ex = jnp.exp(logits - m)
    g = ex / jnp.sum(ex, axis=1, keepdims=True)

    idx8 = lax.broadcasted_iota(jnp.int32, (N, E), 1)
    m1 = jnp.max(g, axis=1, keepdims=True)
    i1 = jnp.min(jnp.where(g == m1, idx8, E), axis=1, keepdims=True)
    hit1 = idx8 == i1
    g2 = jnp.where(hit1, jnp.float32(-1.0), g)
    m2 = jnp.max(g2, axis=1, keepdims=True)
    i2 = jnp.min(jnp.where(g2 == m2, idx8, E), axis=1, keepdims=True)
    hit2 = idx8 == i2

    mask = (hit1 | hit2).astype(jnp.float32)
    s = mask
    d = 1
    while d < N:
        s = s + jnp.concatenate(
            [jnp.zeros((d, E), jnp.float32), s[:-d, :]], axis=0)
        d *= 2
    pos = s - 1.0
    keep = mask * (pos < CAP).astype(jnp.float32)
    w = g * keep

    def pick(hit, arr):
        return jnp.sum(jnp.where(hit, arr, jnp.zeros_like(arr)),
                       axis=1, keepdims=True)

    w0 = pick(hit1, w)
    w1 = pick(hit2, w)
    k0 = pick(hit1, keep)
    k1 = pick(hit2, keep)
    p0 = pick(hit1, pos).astype(jnp.int32)
    p1 = pick(hit2, pos).astype(jnp.int32)
    tid = lax.broadcasted_iota(jnp.int32, (N, 1), 0)
    t0 = SLOTS + ((2 * tid) & (TRASH - 1))
    t1 = SLOTS + ((2 * tid + 1) & (TRASH - 1))
    s0 = jnp.where(k0 > 0, i1 * CAP + p0, t0)
    s1 = jnp.where(k1 > 0, i2 * CAP + p1, t1)
    slots_ref[...] = jnp.concatenate([s0, s1], axis=0).reshape(_NW, _CHUNK)
    wts_ref[pl.ds(0, N), :] = jnp.broadcast_to(w0, (N, WL))
    wts_ref[pl.ds(N, N), :] = jnp.broadcast_to(w1, (N, WL))


_router = pl.pallas_call(
    _router_body,
    out_shape=[
        jax.ShapeDtypeStruct((32, 128), jnp.int32),
        jax.ShapeDtypeStruct((IDX, WL), jnp.float32),
    ],
)


_INV_SQRT2 = 0.7071067811865476


def _ffn_body(xe_ref, w1_ref, b1_ref, w2_ref, b2_ref, ws_ref, out_ref):
    pid = pl.program_id(0)

    @pl.when(pid < E)
    def _():
        xb = xe_ref[...].astype(jnp.bfloat16)
        h = jnp.dot(xb, w1_ref[0].astype(jnp.bfloat16),
                    preferred_element_type=jnp.float32)
        h = h + b1_ref[0]
        h = 0.5 * h * (1.0 + lax.erf(h * _INV_SQRT2))
        part = jnp.dot(h.astype(jnp.bfloat16),
                       w2_ref[0].astype(jnp.bfloat16),
                       preferred_element_type=jnp.float32)
        out_ref[...] = (part + b2_ref[0]) * ws_ref[...][:, 0:1]

    @pl.when(pid == E)
    def _():
        out_ref[...] = jnp.zeros((CAP, D), jnp.float32)


def _clamp_e(e):
    return jnp.minimum(e, E - 1)


_ffn = pl.pallas_call(
    _ffn_body,
    grid=(E + 1,),
    in_specs=[
        pl.BlockSpec((CAP, D), lambda e: (_clamp_e(e), 0)),
        pl.BlockSpec((1, D, H), lambda e: (_clamp_e(e), 0, 0)),
        pl.BlockSpec((1, 1, H), lambda e: (_clamp_e(e), 0, 0)),
        pl.BlockSpec((1, H, D), lambda e: (_clamp_e(e), 0, 0)),
        pl.BlockSpec((1, 1, D), lambda e: (_clamp_e(e), 0, 0)),
        pl.BlockSpec((CAP, WL), lambda e: (_clamp_e(e), 0)),
    ],
    out_specs=pl.BlockSpec((CAP, D), lambda e: (e, 0)),
    out_shape=jax.ShapeDtypeStruct((ROWS, D), jnp.float32),
)


_NW = 32
_CHUNK = IDX // _NW
_TCHUNK = N // _NW


@functools.lru_cache(maxsize=1)
def _sc_kernels():
    mesh = plsc.VectorSubcoreMesh(
        core_axis_name="c", subcore_axis_name="s")
    nc = plsc.get_sparse_core_info().num_cores

    @functools.partial(
        pl.kernel,
        out_type=[
            jax.ShapeDtypeStruct((ROWS, D), jnp.float32),
            jax.ShapeDtypeStruct((ROWS, WL), jnp.float32),
        ],
        mesh=mesh,
        scratch_types=[
            pltpu.VMEM((_TCHUNK,), jnp.int32),
            pltpu.VMEM((_TCHUNK,), jnp.int32),
            pltpu.VMEM((_TCHUNK, D), jnp.float32),
            pltpu.VMEM((_TCHUNK, WL), jnp.float32),
            pltpu.VMEM((_TCHUNK, WL), jnp.float32),
            pltpu.SemaphoreType.DMA,
            pltpu.SemaphoreType.DMA,
        ],
    )
    def dispatch(x_hbm, i_hbm, w_hbm, xe_hbm, ws_hbm,
                 ia_v, ib_v, rows_v, wa_v, wb_v, seml, sems):
        wid = lax.axis_index("s") * nc + lax.axis_index("c")
        tb = wid * _TCHUNK
        l1 = pltpu.async_copy(i_hbm.at[pl.ds(tb, _TCHUNK)], ia_v, seml)
        l2 = pltpu.async_copy(i_hbm.at[pl.ds(N + tb, _TCHUNK)], ib_v, seml)
        l3 = pltpu.async_copy(x_hbm.at[pl.ds(tb, _TCHUNK)], rows_v, seml)
        l4 = pltpu.async_copy(w_hbm.at[pl.ds(tb, _TCHUNK)], wa_v, seml)
        l5 = pltpu.async_copy(w_hbm.at[pl.ds(N + tb, _TCHUNK)], wb_v, seml)
        for c in (l1, l2, l3, l4, l5):
            c.wait()
        c1 = pltpu.async_copy(rows_v, xe_hbm.at[ia_v], sems)
        c2 = pltpu.async_copy(rows_v, xe_hbm.at[ib_v], sems)
        c3 = pltpu.async_copy(wa_v, ws_hbm.at[ia_v], sems)
        c4 = pltpu.async_copy(wb_v, ws_hbm.at[ib_v], sems)
        for c in (c1, c2, c3, c4):
            c.wait()

    @functools.partial(
        pl.kernel,
        out_type=jax.ShapeDtypeStruct((N, D), jnp.float32),
        mesh=mesh,
        scratch_types=[
            pltpu.VMEM((_TCHUNK,), jnp.int32),
            pltpu.VMEM((_TCHUNK,), jnp.int32),
            pltpu.VMEM((_TCHUNK, D), jnp.float32),
            pltpu.VMEM((_TCHUNK, D), jnp.float32),
            pltpu.SemaphoreType.DMA,
            pltpu.SemaphoreType.DMA,
        ],
    )
    def combine(ob_hbm, i_hbm, y_hbm, idx0_v, idx1_v, r0_v, r1_v, s0, s1):
        wid = lax.axis_index("s") * nc + lax.axis_index("c")
        tb = wid * _TCHUNK
        pltpu.sync_copy(i_hbm.at[pl.ds(tb, _TCHUNK)], idx0_v)
        pltpu.sync_copy(i_hbm.at[pl.ds(N + tb, _TCHUNK)], idx1_v)
        c0 = pltpu.async_copy(ob_hbm.at[idx0_v], r0_v, s0)
        c1 = pltpu.async_copy(ob_hbm.at[idx1_v], r1_v, s1)
        c0.wait()
        c1.wait()

        @pl.loop(0, _TCHUNK)
        def _(t):
            for j in range(D // VL):
                sl = pl.ds(j * VL, VL)
                r0_v[t, sl] = r0_v[t, sl] + r1_v[t, sl]

        pltpu.sync_copy(r0_v, y_hbm.at[pl.ds(tb, _TCHUNK)])

    return dispatch, combine


def kernel(x, router_W, router_b, W1, b1, W2, b2):
    x_flat = x.reshape(N, D)
    slots32, wrep = _router(x_flat, router_W, router_b.reshape(1, E))
    slots = slots32.reshape(IDX)
    dispatch, combine = _sc_kernels()
    xe, wslot = dispatch(x_flat, slots, wrep)
    outb = _ffn(xe, W1, b1.reshape(E, 1, H), W2, b2.reshape(E, 1, D), wslot)
    y = combine(outb, slots)
    return y.reshape(B, T, D)


# speedup vs baseline: 6.2011x; 1.0045x over previous
import functools
import math

import jax
import jax.numpy as jnp
from jax import lax
from jax.experimental import pallas as pl
from jax.experimental.pallas import tpu as pltpu
from jax.experimental.pallas import tpu_sc as plsc

B = 1
T = 2048
D = 768
H = 3072
E = 8
CAP = int(math.ceil(T * B / E * 1.25))
N = B * T
SLOTS = E * CAP
TRASH = 256
ROWS = SLOTS + CAP
IDX = 2 * N
WL = 128
VL = 16


def _router_body(x_ref, rw_ref, rb_ref, slots_ref, wts_ref):
    x = x_ref[...]
    logits = lax.dot_general(x, rw_ref[...], (((1,), (1,)), ((), ())),
                             preferred_element_type=jnp.float32)
    logits = logits + rb_ref[...]
    m = jnp.max(logits, axis=1, keepdims=True)
    ex = jnp.exp(logits - m)
    g = ex / jnp.sum(ex, axis=1, keepdims=True)

    idx8 = lax.broadcasted_iota(jnp.int32, (N, E), 1)
    m1 = jnp.max(g, axis=1, keepdims=True)
    i1 = jnp.min(jnp.where(g == m1, idx8, E), axis=1, keepdims=True)
    hit1 = idx8 == i1
    g2 = jnp.where(hit1, jnp.float32(-1.0), g)
    m2 = jnp.max(g2, axis=1, keepdims=True)
    i2 = jnp.min(jnp.where(g2 == m2, idx8, E), axis=1, keepdims=True)
    hit2 = idx8 == i2

    mask = (hit1 | hit2).astype(jnp.float32)
    s = mask
    d = 1
    while d < N:
        s = s + jnp.concatenate(
            [jnp.zeros((d, E), jnp.float32), s[:-d, :]], axis=0)
        d *= 2
    pos = s - 1.0
    keep = mask * (pos < CAP).astype(jnp.float32)
    w = g * keep

    def pick(hit, arr):
        return jnp.sum(jnp.where(hit, arr, jnp.zeros_like(arr)),
                       axis=1, keepdims=True)

    w0 = pick(hit1, w)
    w1 = pick(hit2, w)
    k0 = pick(hit1, keep)
    k1 = pick(hit2, keep)
    p0 = pick(hit1, pos).astype(jnp.int32)
    p1 = pick(hit2, pos).astype(jnp.int32)
    tid = lax.broadcasted_iota(jnp.int32, (N, 1), 0)
    t0 = SLOTS + ((2 * tid) & (TRASH - 1))
    t1 = SLOTS + ((2 * tid + 1) & (TRASH - 1))
    s0 = jnp.where(k0 > 0, i1 * CAP + p0, t0)
    s1 = jnp.where(k1 > 0, i2 * CAP + p1, t1)
    slots_ref[...] = jnp.concatenate([s0, s1], axis=0).reshape(_NW, _CHUNK)
    wts_ref[pl.ds(0, N), :] = jnp.broadcast_to(w0, (N, WL))
    wts_ref[pl.ds(N, N), :] = jnp.broadcast_to(w1, (N, WL))


_router = pl.pallas_call(
    _router_body,
    out_shape=[
        jax.ShapeDtypeStruct((32, 128), jnp.int32),
        jax.ShapeDtypeStruct((IDX, WL), jnp.float32),
    ],
)


_INV_SQRT2 = 0.7071067811865476


def _ffn_body(xe_ref, w1_ref, b1_ref, w2_ref, b2_ref, ws_ref, out_ref):
    pid = pl.program_id(0)

    @pl.when(pid < E)
    def _():
        xb = xe_ref[...].astype(jnp.bfloat16)
        h = jnp.dot(xb, w1_ref[0].astype(jnp.bfloat16),
                    preferred_element_type=jnp.float32)
        h = h + b1_ref[0]
        h = 0.5 * h * (1.0 + lax.erf(h * _INV_SQRT2))
        part = jnp.dot(h.astype(jnp.bfloat16),
                       w2_ref[0].astype(jnp.bfloat16),
                       preferred_element_type=jnp.float32)
        out_ref[...] = (part + b2_ref[0]) * ws_ref[...][:, 0:1]

    @pl.when(pid == E)
    def _():
        out_ref[...] = jnp.zeros((CAP, D), jnp.float32)


def _clamp_e(e):
    return jnp.minimum(e, E - 1)


_ffn = pl.pallas_call(
    _ffn_body,
    grid=(E + 1,),
    in_specs=[
        pl.BlockSpec((CAP, D), lambda e: (_clamp_e(e), 0)),
        pl.BlockSpec((1, D, H), lambda e: (_clamp_e(e), 0, 0)),
        pl.BlockSpec((1, 1, H), lambda e: (_clamp_e(e), 0, 0)),
        pl.BlockSpec((1, H, D), lambda e: (_clamp_e(e), 0, 0)),
        pl.BlockSpec((1, 1, D), lambda e: (_clamp_e(e), 0, 0)),
        pl.BlockSpec((CAP, WL), lambda e: (_clamp_e(e), 0)),
    ],
    out_specs=pl.BlockSpec((CAP, D), lambda e: (e, 0)),
    out_shape=jax.ShapeDtypeStruct((ROWS, D), jnp.float32),
)


_NW = 32
_CHUNK = IDX // _NW
_TCHUNK = N // _NW


@functools.lru_cache(maxsize=1)
def _sc_kernels():
    mesh = plsc.VectorSubcoreMesh(
        core_axis_name="c", subcore_axis_name="s")
    nc = plsc.get_sparse_core_info().num_cores

    @functools.partial(
        pl.kernel,
        out_type=[
            jax.ShapeDtypeStruct((ROWS, D), jnp.float32),
            jax.ShapeDtypeStruct((ROWS, WL), jnp.float32),
        ],
        mesh=mesh,
        scratch_types=[
            pltpu.VMEM((_TCHUNK,), jnp.int32),
            pltpu.VMEM((_TCHUNK,), jnp.int32),
            pltpu.VMEM((_TCHUNK, D), jnp.float32),
            pltpu.VMEM((_TCHUNK, WL), jnp.float32),
            pltpu.VMEM((_TCHUNK, WL), jnp.float32),
            pltpu.SemaphoreType.DMA,
            pltpu.SemaphoreType.DMA,
        ],
    )
    def dispatch(x_hbm, i_hbm, w_hbm, xe_hbm, ws_hbm,
                 ia_v, ib_v, rows_v, wa_v, wb_v, seml, sems):
        wid = lax.axis_index("s") * nc + lax.axis_index("c")
        tb = wid * _TCHUNK
        l1 = pltpu.async_copy(i_hbm.at[pl.ds(tb, _TCHUNK)], ia_v, seml)
        l2 = pltpu.async_copy(i_hbm.at[pl.ds(N + tb, _TCHUNK)], ib_v, seml)
        l3 = pltpu.async_copy(x_hbm.at[pl.ds(tb, _TCHUNK)], rows_v, seml)
        l4 = pltpu.async_copy(w_hbm.at[pl.ds(tb, _TCHUNK)], wa_v, seml)
        l5 = pltpu.async_copy(w_hbm.at[pl.ds(N + tb, _TCHUNK)], wb_v, seml)
        for c in (l1, l2, l3, l4, l5):
            c.wait()
        c1 = pltpu.async_copy(rows_v, xe_hbm.at[ia_v], sems)
        c2 = pltpu.async_copy(rows_v, xe_hbm.at[ib_v], sems)
        c3 = pltpu.async_copy(wa_v, ws_hbm.at[ia_v], sems)
        c4 = pltpu.async_copy(wb_v, ws_hbm.at[ib_v], sems)
        for c in (c1, c2, c3, c4):
            c.wait()

    half = _TCHUNK // 2

    @functools.partial(
        pl.kernel,
        out_type=jax.ShapeDtypeStruct((N, D), jnp.float32),
        mesh=mesh,
        scratch_types=[
            pltpu.VMEM((_TCHUNK,), jnp.int32),
            pltpu.VMEM((_TCHUNK,), jnp.int32),
            pltpu.VMEM((half, D), jnp.float32),
            pltpu.VMEM((half, D), jnp.float32),
            pltpu.VMEM((half, D), jnp.float32),
            pltpu.VMEM((half, D), jnp.float32),
            pltpu.SemaphoreType.DMA,
            pltpu.SemaphoreType.DMA,
            pltpu.SemaphoreType.DMA,
        ],
    )
    def combine(ob_hbm, i_hbm, y_hbm, idx0_v, idx1_v,
                r0a, r1a, r0b, r1b, sa, sb, st):
        wid = lax.axis_index("s") * nc + lax.axis_index("c")
        tb = wid * _TCHUNK
        pltpu.sync_copy(i_hbm.at[pl.ds(tb, _TCHUNK)], idx0_v)
        pltpu.sync_copy(i_hbm.at[pl.ds(N + tb, _TCHUNK)], idx1_v)
        ca0 = pltpu.async_copy(ob_hbm.at[idx0_v.at[pl.ds(0, half)]], r0a, sa)
        ca1 = pltpu.async_copy(ob_hbm.at[idx1_v.at[pl.ds(0, half)]], r1a, sa)
        cb0 = pltpu.async_copy(ob_hbm.at[idx0_v.at[pl.ds(half, half)]], r0b, sb)
        cb1 = pltpu.async_copy(ob_hbm.at[idx1_v.at[pl.ds(half, half)]], r1b, sb)
        ca0.wait()
        ca1.wait()

        @pl.loop(0, half)
        def _(t):
            for j in range(D // VL):
                sl = pl.ds(j * VL, VL)
                r0a[t, sl] = r0a[t, sl] + r1a[t, sl]

        wa = pltpu.async_copy(r0a, y_hbm.at[pl.ds(tb, half)], st)
        cb0.wait()
        cb1.wait()

        @pl.loop(0, half)
        def _(t):
            for j in range(D // VL):
                sl = pl.ds(j * VL, VL)
                r0b[t, sl] = r0b[t, sl] + r1b[t, sl]

        wa.wait()
        pltpu.sync_copy(r0b, y_hbm.at[pl.ds(tb + half, half)])

    return dispatch, combine


def kernel(x, router_W, router_b, W1, b1, W2, b2):
    x_flat = x.reshape(N, D)
    slots32, wrep = _router(x_flat, router_W, router_b.reshape(1, E))
    slots = slots32.reshape(IDX)
    dispatch, combine = _sc_kernels()
    xe, wslot = dispatch(x_flat, slots, wrep)
    outb = _ffn(xe, W1, b1.reshape(E, 1, H), W2, b2.reshape(E, 1, D), wslot)
    y = combine(outb, slots)
    return y.reshape(B, T, D)
